# baseline (speedup 1.0000x reference)
import functools

import jax
import jax.numpy as jnp
from jax import lax
from jax.experimental import pallas as pl
from jax.experimental.pallas import tpu as pltpu
from jax.experimental.pallas import tpu_sc as plsc

_F32 = jnp.float32
_I32 = jnp.int32
_BF = 256
_LANES = 16


def _pad_to(v, m):
    return ((v + m - 1) // m) * m


def _make_agg(V, Vpad, E_pad, nc, with_deg):
    W = _BF // nc
    R = Vpad // 16
    e_per_tile = E_pad // 16
    BK = 128 if nc == 8 else 48
    NBUF = 4
    n_batches = e_per_tile // BK
    n_groups = n_batches // NBUF
    n_pass = nc // 2

    mesh = plsc.VectorSubcoreMesh(core_axis_name="c", subcore_axis_name="s")
    out_type = [jax.ShapeDtypeStruct((Vpad, _BF), _F32)]
    scratch = [
        pltpu.VMEM((NBUF, BK), _I32),
        pltpu.VMEM((NBUF, BK), _I32),
        pltpu.VMEM((NBUF, BK, W), _F32),
        pltpu.VMEM_SHARED((Vpad, W), _F32),
        pltpu.SemaphoreType.DMA((NBUF,)),
        pltpu.SemaphoreType.DMA((NBUF,)),
        pltpu.SemaphoreType.DMA((NBUF,)),
    ]
    n_halves = 2 if nc == 8 else 1
    Vh = Vpad // n_halves
    if with_deg:
        out_type.append(jax.ShapeDtypeStruct((Vpad,), _F32))
        out_type.append(jax.ShapeDtypeStruct((16, Vpad), _F32))
        scratch += [
            pltpu.VMEM((Vh,), _F32),
            pltpu.VMEM((R,), _F32),
            pltpu.VMEM((R,), _F32),
        ]

    def body(x_ref, src_ref, dst_ref, zeros_ref, agg_ref, *rest):
        if with_deg:
            (deg_ref, deg16_ref, srcb, dstb, land, acc, sem_i, sem_g, sem_s,
             degl, degt, dega) = rest
        else:
            srcb, dstb, land, acc, sem_i, sem_g, sem_s = rest
        cid = lax.axis_index("c")
        sid = lax.axis_index("s")

        for ci in range(n_pass):
            c = 2 * ci + cid
            count_deg = with_deg and ci < n_halves
            lo = ci * Vh

            if count_deg:
                @pl.when(cid == 0)
                def _():
                    def zbody(j, _):
                        degl[pl.ds(j * _LANES, _LANES)] = jnp.zeros(
                            (_LANES,), _F32)
                        return 0
                    lax.fori_loop(0, Vh // _LANES, zbody, 0)

            pltpu.sync_copy(zeros_ref, acc.at[pl.ds(sid * R, R), :])
            plsc.subcore_barrier()

            def wait_scatter(b):
                pltpu.make_async_copy(
                    land.at[b], acc.at[pl.ds(0, BK), :], sem_s.at[b]).wait()

            def gbody(g, _):
                for b in range(NBUF):
                    @pl.when(g >= 1)
                    def _(b=b):
                        wait_scatter(b)
                row0 = sid * n_batches + g * NBUF
                pltpu.async_copy(
                    src_ref.at[pl.ds(row0, NBUF), :], srcb, sem_i.at[0])
                pltpu.async_copy(
                    dst_ref.at[pl.ds(row0, NBUF), :], dstb, sem_i.at[1])
                pltpu.make_async_copy(
                    src_ref.at[pl.ds(0, NBUF), :], srcb, sem_i.at[0]).wait()
                pltpu.make_async_copy(
                    dst_ref.at[pl.ds(0, NBUF), :], dstb, sem_i.at[1]).wait()
                for b in range(NBUF):
                    for j in range(BK // _LANES):
                        sl = pl.ds(j * _LANES, _LANES)
                        srcb[b, sl] = srcb[b, sl] * nc + c
                    pltpu.async_copy(
                        x_ref.at[srcb.at[b]], land.at[b], sem_g.at[b])
                if count_deg:
                    @pl.when(cid == 0)
                    def _():
                        ones = jnp.ones((_LANES,), _F32)
                        for b in range(NBUF):
                            for j in range(BK // _LANES):
                                idxv = dstb[b, pl.ds(j * _LANES, _LANES)]
                                if n_halves == 1:
                                    plsc.addupdate_scatter(
                                        degl, [idxv], ones)
                                else:
                                    m = (idxv >= lo) & (idxv < lo + Vh)
                                    plsc.addupdate_scatter(
                                        degl, [idxv - lo], ones, mask=m)
                for b in range(NBUF):
                    pltpu.make_async_copy(
                        x_ref.at[srcb.at[b]], land.at[b], sem_g.at[b]).wait()
                    pltpu.async_copy(
                        land.at[b], acc.at[dstb.at[b]], sem_s.at[b],
                        add=True)
                return 0

            lax.fori_loop(0, n_groups, gbody, 0)
            for b in range(NBUF):
                wait_scatter(b)
            plsc.subcore_barrier()

            if count_deg:
                @pl.when(cid == 0)
                def _():
                    pltpu.sync_copy(degl, deg16_ref.at[sid, pl.ds(lo, Vh)])
                plsc.subcore_barrier()

                @pl.when(jnp.logical_and(
                    cid == 0, sid * n_halves // 16 == ci))
                def _():
                    for t in range(16):
                        pltpu.sync_copy(
                            deg16_ref.at[t, pl.ds(sid * R, R)], degt)
                        def rbody(j, _, _t=t):
                            sl = pl.ds(j * _LANES, _LANES)
                            if _t == 0:
                                dega[sl] = degt[sl]
                            else:
                                dega[sl] = dega[sl] + degt[sl]
                            return 0
                        lax.fori_loop(0, R // _LANES, rbody, 0)
                    pltpu.sync_copy(dega, deg_ref.at[pl.ds(sid * R, R)])

            pltpu.sync_copy(
                acc.at[pl.ds(sid * R, R), :],
                agg_ref.at[pl.ds(sid * R, R), pl.ds(c * W, W)])

    return pl.kernel(body, mesh=mesh, out_type=out_type,
                     scratch_types=scratch,
                     compiler_params=pltpu.CompilerParams(
                         use_tc_tiling_on_sc=False,
                         needs_layout_passes=False))


def _make_pool(Vpad_next):
    rows_per_w = Vpad_next // 32
    n_batches = rows_per_w // 16

    mesh = plsc.VectorSubcoreMesh(core_axis_name="c", subcore_axis_name="s")
    out_type = jax.ShapeDtypeStruct((Vpad_next, _BF), _F32)
    scratch = [
        pltpu.VMEM((112,), _I32),
        pltpu.VMEM((112, _BF), _F32),
        pltpu.VMEM((16, _BF), _F32),
        pltpu.SemaphoreType.DMA,
    ]

    def body(x_ref, idx_ref, out_ref, idxb, land, ob, sem):
        cid = lax.axis_index("c")
        sid = lax.axis_index("s")
        w = sid * 2 + cid

        def pbody(i, _):
            base = w * rows_per_w + i * 16
            pltpu.sync_copy(idx_ref.at[pl.ds(base * 7, 112)], idxb)
            pltpu.async_copy(x_ref.at[idxb], land, sem).wait()

            def rbody(r, _):
                for jc in range(_BF // _LANES):
                    sl = pl.ds(jc * _LANES, _LANES)
                    m = land[r * 7, sl]
                    for k in range(1, 7):
                        m = jnp.maximum(m, land[r * 7 + k, sl])
                    ob[r, sl] = m
                return 0

            lax.fori_loop(0, 16, rbody, 0)
            pltpu.sync_copy(ob, out_ref.at[pl.ds(base, 16), :])
            return 0

        lax.fori_loop(0, n_batches, pbody, 0)

    return pl.kernel(body, mesh=mesh, out_type=out_type,
                     scratch_types=scratch,
                     compiler_params=pltpu.CompilerParams(
                         use_tc_tiling_on_sc=False,
                         needs_layout_passes=False))


_VB = 512


def _sage_pre(V, Vpad):
    grid = (Vpad // _VB,)

    def kfn(x_ref, a_ref, d_ref, ws_ref, wn_ref, b_ref, h_ref, st_ref):
        i = pl.program_id(0)
        x = x_ref[...]
        a = a_ref[...]
        inv = 1.0 / jnp.maximum(d_ref[...], 1.0)
        p = jnp.dot(x, ws_ref[...], preferred_element_type=_F32)
        q = jnp.dot(a, wn_ref[...], preferred_element_type=_F32)
        h = p + q * inv + b_ref[...]
        h_ref[...] = h
        nvalid = V - i * _VB
        rows = lax.broadcasted_iota(_I32, (_VB, _BF), 0)
        hm = jnp.where(rows < nvalid, h, 0.0)
        st = jnp.concatenate(
            [jnp.sum(hm, axis=0, keepdims=True),
             jnp.sum(hm * hm, axis=0, keepdims=True)], axis=0)

        @pl.when(i == 0)
        def _():
            st_ref[...] = jnp.zeros_like(st_ref)

        st_ref[...] += st

    return pl.pallas_call(
        kfn, grid=grid,
        in_specs=[
            pl.BlockSpec((_VB, _BF), lambda i: (i, 0)),
            pl.BlockSpec((_VB, _BF), lambda i: (i, 0)),
            pl.BlockSpec((_VB, 1), lambda i: (i, 0)),
            pl.BlockSpec((_BF, _BF), lambda i: (0, 0)),
            pl.BlockSpec((_BF, _BF), lambda i: (0, 0)),
            pl.BlockSpec((1, _BF), lambda i: (0, 0)),
        ],
        out_specs=[
            pl.BlockSpec((_VB, _BF), lambda i: (i, 0)),
            pl.BlockSpec((2, _BF), lambda i: (0, 0)),
        ],
        out_shape=[
            jax.ShapeDtypeStruct((Vpad, _BF), _F32),
            jax.ShapeDtypeStruct((2, _BF), _F32),
        ])


def _sage_post(V, Vpad):
    grid = (Vpad // _VB,)

    def kfn(h_ref, x_ref, st_ref, g_ref, be_ref, o_ref):
        s = st_ref[...]
        s4 = (s[:, 0:64] + s[:, 64:128] + s[:, 128:192] + s[:, 192:256])
        cnt = 4.0 * V
        mu = s4[0:1] / cnt
        var = s4[1:2] / cnt - mu * mu
        mu4 = jnp.concatenate([mu, mu, mu, mu], axis=1)
        var4 = jnp.concatenate([var, var, var, var], axis=1)
        hn = (h_ref[...] - mu4) * lax.rsqrt(var4 + 1e-5)
        o_ref[...] = (jnp.maximum(g_ref[...] * hn + be_ref[...], 0.0)
                      + x_ref[...])

    return pl.pallas_call(
        kfn, grid=grid,
        in_specs=[
            pl.BlockSpec((_VB, _BF), lambda i: (i, 0)),
            pl.BlockSpec((_VB, _BF), lambda i: (i, 0)),
            pl.BlockSpec((2, _BF), lambda i: (0, 0)),
            pl.BlockSpec((1, _BF), lambda i: (0, 0)),
            pl.BlockSpec((1, _BF), lambda i: (0, 0)),
        ],
        out_specs=pl.BlockSpec((_VB, _BF), lambda i: (i, 0)),
        out_shape=jax.ShapeDtypeStruct((Vpad, _BF), _F32))


def _head(V2, Vpad2):
    def kfn(x_ref, w1_ref, b1_ref, w2_ref, b2_ref, o_ref):
        h = jnp.dot(x_ref[0], w1_ref[...],
                    preferred_element_type=_F32) + b1_ref[...]
        o = jnp.dot(h, w2_ref[...], preferred_element_type=_F32) + b2_ref[...]
        o_ref[...] = o[:V2][None]

    return pl.pallas_call(
        kfn, grid=(4,),
        in_specs=[
            pl.BlockSpec((1, Vpad2, 64), lambda b: (b, 0, 0)),
            pl.BlockSpec((64, 64), lambda b: (0, 0)),
            pl.BlockSpec((1, 64), lambda b: (0, 0)),
            pl.BlockSpec((64, 32), lambda b: (0, 0)),
            pl.BlockSpec((1, 32), lambda b: (0, 0)),
        ],
        out_specs=pl.BlockSpec((1, V2, 32), lambda b: (b, 0, 0)),
        out_shape=jax.ShapeDtypeStruct((4, V2, 32), _F32))


def kernel(features, params, out_params, edges, pools):
    v_levels = [40962, 10242, 2562]
    nc_levels = [8, 2, 2]
    vpad = [_pad_to(v, 512) for v in v_levels]

    x = features.reshape(v_levels[0], _BF)
    x = jnp.concatenate(
        [x, jnp.zeros((vpad[0] - v_levels[0], _BF), _F32)], axis=0)

    eye4 = jnp.eye(4, dtype=_F32)
    for bi in range(3):
        V, Vpad, nc = v_levels[bi], vpad[bi], nc_levels[bi]
        W = _BF // nc
        src, dst = edges[bi]
        E = src.shape[0]
        E_pad = _pad_to(E, 8192 if nc == 8 else 3072)
        bk = 128 if nc == 8 else 48
        srcp = jnp.concatenate(
            [src, jnp.zeros((E_pad - E,), _I32)]).reshape(-1, bk)
        dstp = jnp.concatenate(
            [dst, jnp.full((E_pad - E,), V, _I32)]).reshape(-1, bk)
        zeros_hbm = jnp.zeros((Vpad // 16, W), _F32)

        deg = None
        for li in range(2):
            Ws, Wn, b, g, be = params[bi][li]
            x_view = x.reshape(Vpad * nc, W)
            if li == 0:
                agg, deg, _ = _make_agg(V, Vpad, E_pad, nc, True)(
                    x_view, srcp, dstp, zeros_hbm)
            else:
                (agg,) = _make_agg(V, Vpad, E_pad, nc, False)(
                    x_view, srcp, dstp, zeros_hbm)
            h, st = _sage_pre(V, Vpad)(
                x, agg, deg[:, None], jnp.kron(eye4, Ws), jnp.kron(eye4, Wn),
                jnp.tile(b, 4)[None])
            x = _sage_post(V, Vpad)(
                h, x, st, jnp.tile(g, 4)[None], jnp.tile(be, 4)[None])

        if bi < 2:
            Vn, Vpn = v_levels[bi + 1], vpad[bi + 1]
            base, rest = pools[bi]
            tab = jnp.concatenate(
                [jnp.concatenate([base, base[:, :1]], axis=1), rest], axis=0)
            tab = jnp.concatenate(
                [tab, jnp.zeros((Vpn - Vn, 7), _I32)], axis=0).reshape(-1)
            x = _make_pool(Vpn)(x, tab)

    W1, b1, W2, b2 = out_params
    xt = jnp.transpose(x.reshape(vpad[2], 4, 64), (1, 0, 2))
    return _head(v_levels[2], vpad[2])(xt, W1, b1[None], W2, b2[None])


# speedup vs baseline: 1.0250x; 1.0250x over previous
import functools

import jax
import jax.numpy as jnp
from jax import lax
from jax.experimental import pallas as pl
from jax.experimental.pallas import tpu as pltpu
from jax.experimental.pallas import tpu_sc as plsc

_F32 = jnp.float32
_I32 = jnp.int32
_BF = 256
_LANES = 16


def _pad_to(v, m):
    return ((v + m - 1) // m) * m


def _make_agg(V, Vpad, E_pad, nc, with_deg):
    W = _BF // nc
    R = Vpad // 16
    e_per_tile = E_pad // 16
    BK = 128 if nc == 8 else 48
    NBUF = 4
    n_batches = e_per_tile // BK
    n_groups = n_batches // NBUF
    n_pass = nc // 2

    mesh = plsc.VectorSubcoreMesh(core_axis_name="c", subcore_axis_name="s")
    out_type = [jax.ShapeDtypeStruct((Vpad, _BF), _F32)]
    scratch = [
        pltpu.VMEM((NBUF, BK), _I32),
        pltpu.VMEM((NBUF, BK), _I32),
        pltpu.VMEM((NBUF, BK, W), _F32),
        pltpu.VMEM_SHARED((Vpad, W), _F32),
        pltpu.SemaphoreType.DMA((NBUF,)),
        pltpu.SemaphoreType.DMA((NBUF,)),
        pltpu.SemaphoreType.DMA((NBUF,)),
    ]
    n_halves = 2 if nc == 8 else 1
    Vh = Vpad // n_halves
    if with_deg:
        out_type.append(jax.ShapeDtypeStruct((Vpad,), _F32))
        out_type.append(jax.ShapeDtypeStruct((16, Vpad), _F32))
        scratch += [
            pltpu.VMEM((Vh,), _F32),
            pltpu.VMEM((R,), _F32),
            pltpu.VMEM((R,), _F32),
        ]

    def body(x_ref, src_ref, dst_ref, zeros_ref, agg_ref, *rest):
        if with_deg:
            (deg_ref, deg16_ref, srcb, dstb, land, acc, sem_i, sem_g, sem_s,
             degl, degt, dega) = rest
        else:
            srcb, dstb, land, acc, sem_i, sem_g, sem_s = rest
        cid = lax.axis_index("c")
        sid = lax.axis_index("s")

        for ci in range(n_pass):
            c = 2 * ci + cid
            count_deg = with_deg and ci < n_halves
            lo = ci * Vh

            if count_deg:
                @pl.when(cid == 0)
                def _():
                    def zbody(j, _):
                        degl[pl.ds(j * _LANES, _LANES)] = jnp.zeros(
                            (_LANES,), _F32)
                        return 0
                    lax.fori_loop(0, Vh // _LANES, zbody, 0)

            pltpu.sync_copy(zeros_ref, acc.at[pl.ds(sid * R, R), :])
            plsc.subcore_barrier()

            def wait_scatter(b):
                pltpu.make_async_copy(
                    land.at[b], acc.at[pl.ds(0, BK), :], sem_s.at[b]).wait()

            def fire_idx(i, b):
                pltpu.async_copy(
                    src_ref.at[pl.ds(i, 1), :], srcb.at[pl.ds(b, 1), :],
                    sem_i.at[b])
                pltpu.async_copy(
                    dst_ref.at[pl.ds(i, 1), :], dstb.at[pl.ds(b, 1), :],
                    sem_i.at[b])

            def wait_idx(b):
                pltpu.make_async_copy(
                    src_ref.at[pl.ds(0, 1), :], srcb.at[pl.ds(b, 1), :],
                    sem_i.at[b]).wait()
                pltpu.make_async_copy(
                    dst_ref.at[pl.ds(0, 1), :], dstb.at[pl.ds(b, 1), :],
                    sem_i.at[b]).wait()

            def gbody(g, _):
                for b in range(NBUF):
                    @pl.when(g >= 1)
                    def _(b=b):
                        wait_scatter(b)
                    fire_idx(sid * n_batches + g * NBUF + b, b)
                for b in range(NBUF):
                    wait_idx(b)
                    for j in range(BK // _LANES):
                        sl = pl.ds(j * _LANES, _LANES)
                        srcb[b, sl] = srcb[b, sl] * nc + c
                    pltpu.async_copy(
                        x_ref.at[srcb.at[b]], land.at[b], sem_g.at[b])
                if count_deg:
                    @pl.when(cid == 0)
                    def _():
                        ones = jnp.ones((_LANES,), _F32)
                        for b in range(NBUF):
                            for j in range(BK // _LANES):
                                idxv = dstb[b, pl.ds(j * _LANES, _LANES)]
                                if n_halves == 1:
                                    plsc.addupdate_scatter(
                                        degl, [idxv], ones)
                                else:
                                    m = (idxv >= lo) & (idxv < lo + Vh)
                                    plsc.addupdate_scatter(
                                        degl, [idxv - lo], ones, mask=m)
                for b in range(NBUF):
                    pltpu.make_async_copy(
                        x_ref.at[srcb.at[b]], land.at[b], sem_g.at[b]).wait()
                    pltpu.async_copy(
                        land.at[b], acc.at[dstb.at[b]], sem_s.at[b],
                        add=True)
                return 0

            lax.fori_loop(0, n_groups, gbody, 0)
            for b in range(NBUF):
                wait_scatter(b)
            plsc.subcore_barrier()

            if count_deg:
                @pl.when(cid == 0)
                def _():
                    pltpu.sync_copy(degl, deg16_ref.at[sid, pl.ds(lo, Vh)])
                plsc.subcore_barrier()

                @pl.when(jnp.logical_and(
                    cid == 0, sid * n_halves // 16 == ci))
                def _():
                    for t in range(16):
                        pltpu.sync_copy(
                            deg16_ref.at[t, pl.ds(sid * R, R)], degt)
                        def rbody(j, _, _t=t):
                            sl = pl.ds(j * _LANES, _LANES)
                            if _t == 0:
                                dega[sl] = degt[sl]
                            else:
                                dega[sl] = dega[sl] + degt[sl]
                            return 0
                        lax.fori_loop(0, R // _LANES, rbody, 0)
                    pltpu.sync_copy(dega, deg_ref.at[pl.ds(sid * R, R)])

            pltpu.sync_copy(
                acc.at[pl.ds(sid * R, R), :],
                agg_ref.at[pl.ds(sid * R, R), pl.ds(c * W, W)])

    return pl.kernel(body, mesh=mesh, out_type=out_type,
                     scratch_types=scratch,
                     compiler_params=pltpu.CompilerParams(
                         use_tc_tiling_on_sc=False,
                         needs_layout_passes=False))


def _make_pool(Vpad_next):
    rows_per_w = Vpad_next // 32
    n_batches = rows_per_w // 16
    NB2 = 2

    mesh = plsc.VectorSubcoreMesh(core_axis_name="c", subcore_axis_name="s")
    out_type = jax.ShapeDtypeStruct((Vpad_next, _BF), _F32)
    scratch = [
        pltpu.VMEM((NB2, 112), _I32),
        pltpu.VMEM((NB2, 112, _BF), _F32),
        pltpu.VMEM((NB2, 16, _BF), _F32),
        pltpu.SemaphoreType.DMA((NB2,)),
        pltpu.SemaphoreType.DMA((NB2,)),
        pltpu.SemaphoreType.DMA((NB2,)),
    ]

    def body(x_ref, idx_ref, out_ref, idxb, land, ob, sem_i, sem_g, sem_o):
        cid = lax.axis_index("c")
        sid = lax.axis_index("s")
        w = sid * 2 + cid

        def do_batch(i, b, g):
            pltpu.make_async_copy(
                x_ref.at[idxb.at[b]], land.at[b], sem_g.at[b]).wait()

            def rbody(r, _, b=b):
                for jc in range(_BF // _LANES):
                    sl = pl.ds(jc * _LANES, _LANES)
                    m = land[b, r * 7, sl]
                    for k in range(1, 7):
                        m = jnp.maximum(m, land[b, r * 7 + k, sl])
                    ob[b, r, sl] = m
                return 0

            lax.fori_loop(0, 16, rbody, 0)
            base = w * rows_per_w + i * 16
            pltpu.async_copy(
                ob.at[b], out_ref.at[pl.ds(base, 16), :], sem_o.at[b])

        def fire(i, b, g):
            @pl.when(g >= 1)
            def _():
                pltpu.make_async_copy(
                    ob.at[b], out_ref.at[pl.ds(0, 16), :], sem_o.at[b]).wait()
            base = w * rows_per_w + i * 16
            pltpu.async_copy(
                idx_ref.at[pl.ds(base * 7, 112)], idxb.at[b], sem_i.at[b])
            pltpu.make_async_copy(
                idx_ref.at[pl.ds(0, 112)], idxb.at[b], sem_i.at[b]).wait()
            pltpu.async_copy(x_ref.at[idxb.at[b]], land.at[b], sem_g.at[b])

        n_groups = n_batches // NB2

        def pbody(g, _):
            for b in range(NB2):
                fire(g * NB2 + b, b, g)
            for b in range(NB2):
                do_batch(g * NB2 + b, b, g)
            return 0

        lax.fori_loop(0, n_groups, pbody, 0)
        if n_batches % NB2:
            i = n_groups * NB2
            fire(i, 0, n_groups)
            do_batch(i, 0, n_groups)
            for b in range(1, NB2):
                pltpu.make_async_copy(
                    ob.at[b], out_ref.at[pl.ds(0, 16), :], sem_o.at[b]).wait()
            pltpu.make_async_copy(
                ob.at[0], out_ref.at[pl.ds(0, 16), :], sem_o.at[0]).wait()
        else:
            for b in range(NB2):
                pltpu.make_async_copy(
                    ob.at[b], out_ref.at[pl.ds(0, 16), :], sem_o.at[b]).wait()

    return pl.kernel(body, mesh=mesh, out_type=out_type,
                     scratch_types=scratch,
                     compiler_params=pltpu.CompilerParams(
                         use_tc_tiling_on_sc=False,
                         needs_layout_passes=False))


_VB = 512


def _sage_pre(V, Vpad):
    grid = (Vpad // _VB,)

    def kfn(x_ref, a_ref, d_ref, ws_ref, wn_ref, b_ref, h_ref, st_ref):
        i = pl.program_id(0)
        x = x_ref[...]
        a = a_ref[...]
        inv = 1.0 / jnp.maximum(d_ref[...], 1.0)
        p = jnp.dot(x, ws_ref[...], preferred_element_type=_F32)
        q = jnp.dot(a, wn_ref[...], preferred_element_type=_F32)
        h = p + q * inv + b_ref[...]
        h_ref[...] = h
        nvalid = V - i * _VB
        rows = lax.broadcasted_iota(_I32, (_VB, _BF), 0)
        hm = jnp.where(rows < nvalid, h, 0.0)
        st = jnp.concatenate(
            [jnp.sum(hm, axis=0, keepdims=True),
             jnp.sum(hm * hm, axis=0, keepdims=True)], axis=0)

        @pl.when(i == 0)
        def _():
            st_ref[...] = jnp.zeros_like(st_ref)

        st_ref[...] += st

    return pl.pallas_call(
        kfn, grid=grid,
        in_specs=[
            pl.BlockSpec((_VB, _BF), lambda i: (i, 0)),
            pl.BlockSpec((_VB, _BF), lambda i: (i, 0)),
            pl.BlockSpec((_VB, 1), lambda i: (i, 0)),
            pl.BlockSpec((_BF, _BF), lambda i: (0, 0)),
            pl.BlockSpec((_BF, _BF), lambda i: (0, 0)),
            pl.BlockSpec((1, _BF), lambda i: (0, 0)),
        ],
        out_specs=[
            pl.BlockSpec((_VB, _BF), lambda i: (i, 0)),
            pl.BlockSpec((2, _BF), lambda i: (0, 0)),
        ],
        out_shape=[
            jax.ShapeDtypeStruct((Vpad, _BF), _F32),
            jax.ShapeDtypeStruct((2, _BF), _F32),
        ])


def _sage_post(V, Vpad):
    grid = (Vpad // _VB,)

    def kfn(h_ref, x_ref, st_ref, g_ref, be_ref, o_ref):
        s = st_ref[...]
        s4 = (s[:, 0:64] + s[:, 64:128] + s[:, 128:192] + s[:, 192:256])
        cnt = 4.0 * V
        mu = s4[0:1] / cnt
        var = s4[1:2] / cnt - mu * mu
        mu4 = jnp.concatenate([mu, mu, mu, mu], axis=1)
        var4 = jnp.concatenate([var, var, var, var], axis=1)
        hn = (h_ref[...] - mu4) * lax.rsqrt(var4 + 1e-5)
        o_ref[...] = (jnp.maximum(g_ref[...] * hn + be_ref[...], 0.0)
                      + x_ref[...])

    return pl.pallas_call(
        kfn, grid=grid,
        in_specs=[
            pl.BlockSpec((_VB, _BF), lambda i: (i, 0)),
            pl.BlockSpec((_VB, _BF), lambda i: (i, 0)),
            pl.BlockSpec((2, _BF), lambda i: (0, 0)),
            pl.BlockSpec((1, _BF), lambda i: (0, 0)),
            pl.BlockSpec((1, _BF), lambda i: (0, 0)),
        ],
        out_specs=pl.BlockSpec((_VB, _BF), lambda i: (i, 0)),
        out_shape=jax.ShapeDtypeStruct((Vpad, _BF), _F32))


def _head(V2, Vpad2):
    def kfn(x_ref, w1_ref, b1_ref, w2_ref, b2_ref, o_ref):
        h = jnp.dot(x_ref[0], w1_ref[...],
                    preferred_element_type=_F32) + b1_ref[...]
        o = jnp.dot(h, w2_ref[...], preferred_element_type=_F32) + b2_ref[...]
        o_ref[...] = o[:V2][None]

    return pl.pallas_call(
        kfn, grid=(4,),
        in_specs=[
            pl.BlockSpec((1, Vpad2, 64), lambda b: (b, 0, 0)),
            pl.BlockSpec((64, 64), lambda b: (0, 0)),
            pl.BlockSpec((1, 64), lambda b: (0, 0)),
            pl.BlockSpec((64, 32), lambda b: (0, 0)),
            pl.BlockSpec((1, 32), lambda b: (0, 0)),
        ],
        out_specs=pl.BlockSpec((1, V2, 32), lambda b: (b, 0, 0)),
        out_shape=jax.ShapeDtypeStruct((4, V2, 32), _F32))


def kernel(features, params, out_params, edges, pools):
    v_levels = [40962, 10242, 2562]
    nc_levels = [8, 2, 2]
    vpad = [_pad_to(v, 512) for v in v_levels]

    x = features.reshape(v_levels[0], _BF)
    x = jnp.concatenate(
        [x, jnp.zeros((vpad[0] - v_levels[0], _BF), _F32)], axis=0)

    eye4 = jnp.eye(4, dtype=_F32)
    for bi in range(3):
        V, Vpad, nc = v_levels[bi], vpad[bi], nc_levels[bi]
        W = _BF // nc
        src, dst = edges[bi]
        E = src.shape[0]
        E_pad = _pad_to(E, 8192 if nc == 8 else 3072)
        bk = 128 if nc == 8 else 48
        srcp = jnp.concatenate(
            [src, jnp.zeros((E_pad - E,), _I32)]).reshape(-1, bk)
        dstp = jnp.concatenate(
            [dst, jnp.full((E_pad - E,), V, _I32)]).reshape(-1, bk)
        zeros_hbm = jnp.zeros((Vpad // 16, W), _F32)

        deg = None
        for li in range(2):
            Ws, Wn, b, g, be = params[bi][li]
            x_view = x.reshape(Vpad * nc, W)
            if li == 0:
                agg, deg, _ = _make_agg(V, Vpad, E_pad, nc, True)(
                    x_view, srcp, dstp, zeros_hbm)
            else:
                (agg,) = _make_agg(V, Vpad, E_pad, nc, False)(
                    x_view, srcp, dstp, zeros_hbm)
            h, st = _sage_pre(V, Vpad)(
                x, agg, deg[:, None], jnp.kron(eye4, Ws), jnp.kron(eye4, Wn),
                jnp.tile(b, 4)[None])
            x = _sage_post(V, Vpad)(
                h, x, st, jnp.tile(g, 4)[None], jnp.tile(be, 4)[None])

        if bi < 2:
            Vn, Vpn = v_levels[bi + 1], vpad[bi + 1]
            base, rest = pools[bi]
            tab = jnp.concatenate(
                [jnp.concatenate([base, base[:, :1]], axis=1), rest], axis=0)
            tab = jnp.concatenate(
                [tab, jnp.zeros((Vpn - Vn, 7), _I32)], axis=0).reshape(-1)
            x = _make_pool(Vpn)(x, tab)

    W1, b1, W2, b2 = out_params
    xt = jnp.transpose(x.reshape(vpad[2], 4, 64), (1, 0, 2))
    return _head(v_levels[2], vpad[2])(xt, W1, b1[None], W2, b2[None])
